# Initial kernel scaffold; baseline (speedup 1.0000x reference)
import jax
import jax.numpy as jnp
from jax.experimental import pallas as pl


def kernel(x, edge_index, W, b, gamma, beta):
    raise NotImplementedError("write your pallas kernel here")


# speedup vs baseline: 20.3348x; 20.3348x over previous
import functools

import jax
import jax.numpy as jnp
from jax import lax
from jax.experimental import pallas as pl
from jax.experimental.pallas import tpu as pltpu
from jax.experimental.pallas import tpu_sc as plsc

N = 10000
D = 128
E = 320000
NC = 2
NS = 16
NW = NC * NS
CHUNK = 128
NP = 10112
ROWS_PER_TILE = NP // NS
E_TOT = E + N
CH = -(-E_TOT // (NW * CHUNK))
EP = NW * CH * CHUNK
TRASH = N
ZROW = N + 1

_mesh = plsc.VectorSubcoreMesh(
    core_axis_name="c", subcore_axis_name="s", num_cores=NC, num_subcores=NS
)


@functools.partial(
    pl.kernel,
    out_type=jax.ShapeDtypeStruct((NC, NP, 16), jnp.float32),
    mesh=_mesh,
    scratch_types=[
        pltpu.VMEM_SHARED((NP, 16), jnp.float32),
        pltpu.VMEM((CH, CHUNK), jnp.int32),
        pltpu.VMEM((CHUNK, 16), jnp.float32),
    ],
)
def _deg_kernel(dst_hbm, zeros_hbm, ones_hbm, out_hbm, degw_sh, dstv, ones_v):
    c = lax.axis_index("c")
    s = lax.axis_index("s")
    wid = s * NC + c
    pltpu.sync_copy(
        zeros_hbm.at[pl.ds(s * ROWS_PER_TILE, ROWS_PER_TILE)],
        degw_sh.at[pl.ds(s * ROWS_PER_TILE, ROWS_PER_TILE)],
    )
    pltpu.sync_copy(ones_hbm, ones_v)
    pltpu.sync_copy(dst_hbm.at[wid], dstv)
    plsc.subcore_barrier()

    @pl.loop(0, CH)
    def _(j):
        pltpu.sync_copy(ones_v, degw_sh.at[dstv.at[j]], add=True)

    plsc.subcore_barrier()
    pltpu.sync_copy(
        degw_sh.at[pl.ds(s * ROWS_PER_TILE, ROWS_PER_TILE)],
        out_hbm.at[c, pl.ds(s * ROWS_PER_TILE, ROWS_PER_TILE)],
    )


def _matmul_body(x_ref, w_ref, degw_ref, o_ref):
    h = jnp.dot(x_ref[...], w_ref[...], preferred_element_type=jnp.float32)
    deg = degw_ref[0, :, :1] + degw_ref[1, :, :1]
    dis = jnp.where(deg > 0.0, lax.rsqrt(deg), 0.0)
    o_ref[...] = h * dis


_matmul = pl.pallas_call(
    _matmul_body,
    grid=(NP // 128,),
    in_specs=[
        pl.BlockSpec((128, D), lambda j: (j, 0)),
        pl.BlockSpec((D, D), lambda j: (0, 0)),
        pl.BlockSpec((NC, 128, 16), lambda j: (0, j, 0)),
    ],
    out_specs=pl.BlockSpec((128, D), lambda j: (j, 0)),
    out_shape=jax.ShapeDtypeStruct((NP, D), jnp.float32),
)


@functools.partial(
    pl.kernel,
    out_type=jax.ShapeDtypeStruct((NC, NP, D), jnp.float32),
    mesh=_mesh,
    scratch_types=[
        pltpu.VMEM_SHARED((NP, D), jnp.float32),
        pltpu.VMEM((CH, CHUNK), jnp.int32),
        pltpu.VMEM((CH, CHUNK), jnp.int32),
        pltpu.VMEM((CHUNK, D), jnp.float32),
        pltpu.SemaphoreType.DMA,
    ],
)
def _agg_kernel(h_hbm, src_hbm, dst_hbm, zeros_hbm, out_hbm,
                acc_sh, srcv, dstv, stag, gsem):
    c = lax.axis_index("c")
    s = lax.axis_index("s")
    wid = s * NC + c
    pltpu.sync_copy(
        zeros_hbm.at[pl.ds(s * ROWS_PER_TILE, ROWS_PER_TILE)],
        acc_sh.at[pl.ds(s * ROWS_PER_TILE, ROWS_PER_TILE)],
    )
    pltpu.sync_copy(src_hbm.at[wid], srcv)
    pltpu.sync_copy(dst_hbm.at[wid], dstv)
    plsc.subcore_barrier()

    @pl.loop(0, CH)
    def _(j):
        pltpu.async_copy(h_hbm.at[srcv.at[j]], stag, gsem).wait()
        pltpu.sync_copy(stag, acc_sh.at[dstv.at[j]], add=True)

    plsc.subcore_barrier()
    pltpu.sync_copy(
        acc_sh.at[pl.ds(s * ROWS_PER_TILE, ROWS_PER_TILE)],
        out_hbm.at[c, pl.ds(s * ROWS_PER_TILE, ROWS_PER_TILE)],
    )


def _final_body(acc_ref, degw_ref, b_ref, gamma_ref, beta_ref, o_ref):
    a = acc_ref[0, :N, :] + acc_ref[1, :N, :]
    deg = degw_ref[0, :N, :1] + degw_ref[1, :N, :1]
    pre = a * lax.rsqrt(deg) + b_ref[...]
    mean = jnp.mean(pre, axis=0, keepdims=True)
    var = jnp.mean((pre - mean) * (pre - mean), axis=0, keepdims=True)
    o = (pre - mean) * lax.rsqrt(var + 1e-5) * gamma_ref[...] + beta_ref[...]
    o_ref[...] = jnp.maximum(o, 0.0)


_final = pl.pallas_call(
    _final_body,
    out_shape=jax.ShapeDtypeStruct((N, D), jnp.float32),
)


def kernel(x, edge_index, W, b, gamma, beta):
    loop_idx = jnp.arange(N, dtype=jnp.int32)
    pad = EP - E_TOT
    src_all = jnp.concatenate(
        [edge_index[0], loop_idx, jnp.full((pad,), ZROW, jnp.int32)]
    ).reshape(NW, CH, CHUNK)
    dst_all = jnp.concatenate(
        [edge_index[1], loop_idx, jnp.full((pad,), TRASH, jnp.int32)]
    ).reshape(NW, CH, CHUNK)
    x_pad = jnp.concatenate([x, jnp.zeros((NP - N, D), jnp.float32)])

    degw = _deg_kernel(
        dst_all, jnp.zeros((NP, 16), jnp.float32), jnp.ones((CHUNK, 16), jnp.float32)
    )
    h_t = _matmul(x_pad, W, degw)
    acc = _agg_kernel(h_t, src_all, dst_all, jnp.zeros((NP, D), jnp.float32))
    return _final(
        acc, degw, b.reshape(1, D), gamma.reshape(1, D), beta.reshape(1, D)
    )
